# baseline (speedup 1.0000x reference)
import functools

import jax
import jax.numpy as jnp
from jax import lax
from jax.experimental import pallas as pl
from jax.experimental.pallas import tpu as pltpu
from jax.experimental.pallas import tpu_sc as plsc

_N_CELLS = 200
_N_REGIONS = 500
_N_FRAGMENTS = 6400000
_NUM_SEG = _N_CELLS * _N_REGIONS

_NW = 32
_PER_W = _N_FRAGMENTS // _NW
_CHUNK = 2000
_NBUF = 4
_NCH = _PER_W // _CHUNK
_NV = _CHUNK // 16
_UNROLL = 5

_mesh = plsc.VectorSubcoreMesh(core_axis_name="c", subcore_axis_name="s")


@functools.partial(
    pl.kernel,
    out_type=jax.ShapeDtypeStruct((_NW, _NUM_SEG), jnp.float32),
    mesh=_mesh,
    compiler_params=pltpu.CompilerParams(needs_layout_passes=False),
    scratch_types=[
        pltpu.VMEM((_NUM_SEG,), jnp.float32),
        pltpu.VMEM((_NBUF * _CHUNK + 16,), jnp.int32),
        pltpu.VMEM((_NBUF * _CHUNK,), jnp.float32),
        pltpu.SemaphoreType.DMA((_NBUF,)),
        pltpu.SemaphoreType.DMA((_NBUF,)),
    ],
)
def _sc_segsum(ix_hbm, pos_hbm, out_hbm, acc, ixb, valb, six, sval):
    wid = lax.axis_index("s") * 2 + lax.axis_index("c")
    base = wid * _PER_W

    zero16 = jnp.zeros((16,), jnp.float32)

    def zbody(i, carry):
        for k in range(8):
            acc[pl.ds(i * 128 + k * 16, 16)] = zero16
        return carry

    lax.fori_loop(0, _NUM_SEG // 128, zbody, 0)
    acc[pl.ds(_NUM_SEG - 32, 16)] = zero16
    acc[pl.ds(_NUM_SEG - 16, 16)] = zero16

    lane = lax.iota(jnp.int32, 16)
    lane15 = lane == 15
    not15 = lane != 15

    def start_chunk(c, slot):
        off = base + c * _CHUNK
        pltpu.async_copy(
            ix_hbm.at[pl.ds(off, _CHUNK)], ixb.at[pl.ds(slot * _CHUNK, _CHUNK)],
            six.at[slot])
        pltpu.async_copy(
            pos_hbm.at[pl.ds(off, _CHUNK)],
            valb.at[pl.ds(slot * _CHUNK, _CHUNK)], sval.at[slot])

    def wait_chunk(c, slot):
        off = base + c * _CHUNK
        pltpu.make_async_copy(
            ix_hbm.at[pl.ds(off, _CHUNK)], ixb.at[pl.ds(slot * _CHUNK, _CHUNK)],
            six.at[slot]).wait()
        pltpu.make_async_copy(
            pos_hbm.at[pl.ds(off, _CHUNK)],
            valb.at[pl.ds(slot * _CHUNK, _CHUNK)], sval.at[slot]).wait()

    def group_step(ib, vb, g):
        items = []
        for u in range(_UNROLL):
            j = g * _UNROLL + u
            idx = ixb[pl.ds(ib + j * 16, 16)]
            idxn = ixb[pl.ds(ib + j * 16 + 1, 16)]
            val = valb[pl.ds(vb + j * 16, 16)]
            items.append((idx, idxn, val))
        outs = []
        for idx, idxn, val in items:
            c = plsc.cumsum(val)
            bound = idx != idxn
            outs.append((idx, idxn, c, bound))
        for idx, idxn, c, bound in outs:
            plsc.addupdate_scatter(acc, [idx], c, mask=bound | lane15)
            plsc.addupdate_scatter(acc, [idxn], -c, mask=bound & not15)

    def process_chunk(slot):
        ib = slot * _CHUNK
        vb = slot * _CHUNK

        def vbody(g, carry):
            group_step(ib, vb, g)
            return carry

        lax.fori_loop(0, _NV // _UNROLL, vbody, 0)

    for p in range(_NBUF - 1):
        start_chunk(p, p)

    def outer(c, carry):
        slot = lax.rem(c, _NBUF)

        @pl.when(c + _NBUF - 1 < _NCH)
        def _():
            start_chunk(c + _NBUF - 1, lax.rem(c + _NBUF - 1, _NBUF))

        wait_chunk(c, slot)
        process_chunk(slot)
        return carry

    lax.fori_loop(0, _NCH, outer, 0)

    pltpu.sync_copy(acc, out_hbm.at[wid])


def _tc_combine(parts_ref, cnt_ref, out_ref):
    out_ref[...] = jnp.sum(parts_ref[...], axis=0) + cnt_ref[...]


def kernel(likelihood_position, likelihood_count, local_cellxregion_ix):
    ix = local_cellxregion_ix.astype(jnp.int32)
    parts = _sc_segsum(ix, likelihood_position)
    parts3 = parts.reshape(_NW, _N_CELLS, _N_REGIONS)
    cnt = likelihood_count.reshape(_N_CELLS, _N_REGIONS)
    nblk = 5
    blk = _N_CELLS // nblk
    out = pl.pallas_call(
        _tc_combine,
        grid=(nblk,),
        in_specs=[
            pl.BlockSpec((_NW, blk, _N_REGIONS), lambda i: (0, i, 0)),
            pl.BlockSpec((blk, _N_REGIONS), lambda i: (i, 0)),
        ],
        out_specs=pl.BlockSpec((blk, _N_REGIONS), lambda i: (i, 0)),
        out_shape=jax.ShapeDtypeStruct((_N_CELLS, _N_REGIONS), jnp.float32),
    )(parts3, cnt)
    return out


# speedup vs baseline: 1.0178x; 1.0178x over previous
import functools

import jax
import jax.numpy as jnp
from jax import lax
from jax.experimental import pallas as pl
from jax.experimental.pallas import tpu as pltpu
from jax.experimental.pallas import tpu_sc as plsc

_N_CELLS = 200
_N_REGIONS = 500
_N_FRAGMENTS = 6400000
_NUM_SEG = _N_CELLS * _N_REGIONS

_NW = 32
_PER_W = _N_FRAGMENTS // _NW
_CHUNK = 2000
_NBUF = 4
_NCH = _PER_W // _CHUNK
_NV = _CHUNK // 16
_UNROLL = 5

_mesh = plsc.VectorSubcoreMesh(core_axis_name="c", subcore_axis_name="s")


@functools.partial(
    pl.kernel,
    out_type=jax.ShapeDtypeStruct((_NW, _NUM_SEG), jnp.float32),
    mesh=_mesh,
    compiler_params=pltpu.CompilerParams(needs_layout_passes=False),
    scratch_types=[
        pltpu.VMEM((_NUM_SEG,), jnp.float32),
        pltpu.VMEM((_NBUF * _CHUNK + 16,), jnp.int32),
        pltpu.VMEM((_NBUF * _CHUNK,), jnp.float32),
        pltpu.SemaphoreType.DMA((_NBUF,)),
        pltpu.SemaphoreType.DMA((_NBUF,)),
    ],
)
def _sc_segsum(ix_hbm, pos_hbm, out_hbm, acc, ixb, valb, six, sval):
    wid = lax.axis_index("s") * 2 + lax.axis_index("c")
    base = wid * _PER_W

    zero16 = jnp.zeros((16,), jnp.float32)

    lane = lax.iota(jnp.int32, 16)
    lane15 = lane == 15
    not15 = lane != 15

    def start_chunk(c, slot):
        off = base + c * _CHUNK
        pltpu.async_copy(
            ix_hbm.at[pl.ds(off, _CHUNK)], ixb.at[pl.ds(slot * _CHUNK, _CHUNK)],
            six.at[slot])
        pltpu.async_copy(
            pos_hbm.at[pl.ds(off, _CHUNK)],
            valb.at[pl.ds(slot * _CHUNK, _CHUNK)], sval.at[slot])

    def wait_chunk(c, slot):
        off = base + c * _CHUNK
        pltpu.make_async_copy(
            ix_hbm.at[pl.ds(off, _CHUNK)], ixb.at[pl.ds(slot * _CHUNK, _CHUNK)],
            six.at[slot]).wait()
        pltpu.make_async_copy(
            pos_hbm.at[pl.ds(off, _CHUNK)],
            valb.at[pl.ds(slot * _CHUNK, _CHUNK)], sval.at[slot]).wait()

    for p in range(_NBUF - 1):
        start_chunk(p, p)

    def zbody(i, carry):
        for k in range(8):
            acc[pl.ds(i * 128 + k * 16, 16)] = zero16
        return carry

    lax.fori_loop(0, _NUM_SEG // 128, zbody, 0)
    acc[pl.ds(_NUM_SEG - 32, 16)] = zero16
    acc[pl.ds(_NUM_SEG - 16, 16)] = zero16

    def group_step(ib, vb, g):
        items = []
        for u in range(_UNROLL):
            j = g * _UNROLL + u
            idx = ixb[pl.ds(ib + j * 16, 16)]
            idxn = ixb[pl.ds(ib + j * 16 + 1, 16)]
            val = valb[pl.ds(vb + j * 16, 16)]
            items.append((idx, idxn, val))
        outs = []
        for idx, idxn, val in items:
            c = plsc.cumsum(val)
            bound = idx != idxn
            outs.append((idx, idxn, c, bound))
        for idx, idxn, c, bound in outs:
            plsc.addupdate_scatter(acc, [idx], c, mask=bound | lane15)
            plsc.addupdate_scatter(acc, [idxn], -c, mask=bound & not15)

    def process_chunk(slot):
        ib = slot * _CHUNK
        vb = slot * _CHUNK

        def vbody(g, carry):
            group_step(ib, vb, g)
            return carry

        lax.fori_loop(0, _NV // _UNROLL, vbody, 0)

    def outer(c, carry):
        slot = lax.rem(c, _NBUF)

        @pl.when(c + _NBUF - 1 < _NCH)
        def _():
            start_chunk(c + _NBUF - 1, lax.rem(c + _NBUF - 1, _NBUF))

        wait_chunk(c, slot)
        process_chunk(slot)
        return carry

    lax.fori_loop(0, _NCH, outer, 0)

    pltpu.sync_copy(acc, out_hbm.at[wid])


def _tc_combine(parts_ref, cnt_ref, out_ref):
    out_ref[...] = jnp.sum(parts_ref[...], axis=0) + cnt_ref[...]


def kernel(likelihood_position, likelihood_count, local_cellxregion_ix):
    ix = local_cellxregion_ix.astype(jnp.int32)
    parts = _sc_segsum(ix, likelihood_position)
    parts3 = parts.reshape(_NW, _N_CELLS, _N_REGIONS)
    cnt = likelihood_count.reshape(_N_CELLS, _N_REGIONS)
    out = pl.pallas_call(
        _tc_combine,
        out_shape=jax.ShapeDtypeStruct((_N_CELLS, _N_REGIONS), jnp.float32),
    )(parts3, cnt)
    return out
